# Initial kernel scaffold; baseline (speedup 1.0000x reference)
import jax
import jax.numpy as jnp
import numpy as np
from jax import lax
from jax.experimental import pallas as pl
from jax.experimental.pallas import tpu as pltpu
from jax.experimental.pallas import tpu_sc as plsc

B = 256
NPG = 4096
N = B * NPG
D = 64
H = 128
NB = 65536
FMIN = float(np.finfo(np.float32).min)
FMAX = float(np.finfo(np.float32).max)

NWORKERS = 32
REGION = N // NWORKERS


def _sc_mask_body(banned_ref, fill_ref, out_ref, idx_v, region_v):
    cid = lax.axis_index("c")
    sid = lax.axis_index("s")
    wid = sid * 2 + cid
    lo = wid * REGION
    pltpu.sync_copy(fill_ref, region_v)
    pltpu.sync_copy(banned_ref, idx_v)
    fminv = jnp.full((16,), FMIN, jnp.float32)

    def body(i, carry):
        v = idx_v[pl.ds(i * 16, 16)]
        lcl = v - lo
        m = (lcl >= 0) & (lcl < REGION)
        lc = jnp.clip(lcl, 0, REGION - 1)
        plsc.store_scatter(region_v, [lc], fminv, mask=m)
        return carry

    lax.fori_loop(0, NB // 16, body, 0, unroll=4)
    pltpu.sync_copy(region_v, out_ref.at[pl.ds(lo, REGION)])


def _sc_mask(banned_idx, fill):
    return pl.kernel(
        _sc_mask_body,
        out_type=jax.ShapeDtypeStruct((N,), jnp.float32),
        mesh=plsc.VectorSubcoreMesh(core_axis_name="c", subcore_axis_name="s"),
        scratch_types=[
            pltpu.VMEM((NB,), jnp.int32),
            pltpu.VMEM((REGION,), jnp.float32),
        ],
    )(banned_idx, fill)


def _tc_body(x_ref, g_ref, m_ref, w1_ref, b1_ref, w2_ref, b2_ref,
             act_ref, val_ref):
    x = x_ref[...]
    g = g_ref[0]
    c = jnp.concatenate([x, jnp.broadcast_to(g, (NPG, D))], axis=1)
    h = lax.dot_general(c, w1_ref[...], (((1,), (0,)), ((), ())),
                        preferred_element_type=jnp.float32)
    h = jnp.maximum(h + b1_ref[...], 0.0)
    q = lax.dot_general(w2_ref[...], h, (((1,), (1,)), ((), ())),
                        preferred_element_type=jnp.float32)
    q = q + b2_ref[0, 0]
    qm = jnp.minimum(q, m_ref[0])
    vmax = jnp.max(qm)
    ids = lax.broadcasted_iota(jnp.int32, (1, NPG), 1)
    cand = jnp.where(qm == vmax, ids, NPG)
    amin = jnp.min(cand)
    act_ref[...] = jnp.reshape(amin, (1, 1, 1))
    val_ref[...] = jnp.reshape(vmax, (1, 1, 1))


def _tc_call(node_embed, gemb3, mask3, W1, b1r, w2r, b2s):
    acts, vals = pl.pallas_call(
        _tc_body,
        grid=(B,),
        in_specs=[
            pl.BlockSpec((NPG, D), lambda g: (g, 0)),
            pl.BlockSpec((1, 1, D), lambda g: (g, 0, 0)),
            pl.BlockSpec((1, 1, NPG), lambda g: (g, 0, 0)),
            pl.BlockSpec((2 * D, H), lambda g: (0, 0)),
            pl.BlockSpec((1, H), lambda g: (0, 0)),
            pl.BlockSpec((1, H), lambda g: (0, 0)),
            pl.BlockSpec(memory_space=pltpu.SMEM),
        ],
        out_specs=[
            pl.BlockSpec((1, 1, 1), lambda g: (g, 0, 0)),
            pl.BlockSpec((1, 1, 1), lambda g: (g, 0, 0)),
        ],
        out_shape=[
            jax.ShapeDtypeStruct((B, 1, 1), jnp.int32),
            jax.ShapeDtypeStruct((B, 1, 1), jnp.float32),
        ],
        compiler_params=pltpu.CompilerParams(
            dimension_semantics=("arbitrary",),
        ),
    )(node_embed, gemb3, mask3, W1, b1r, w2r, b2s)
    return acts, vals


def kernel(node_embed, graph_embed, prefix_sum, banned_idx, W1, b1, W2, b2):
    fill = jnp.full((REGION,), FMAX, jnp.float32)
    mask = _sc_mask(banned_idx, fill)
    mask3 = mask.reshape(B, 1, NPG)
    gemb3 = graph_embed.reshape(B, 1, D)
    b1r = b1.reshape(1, H)
    w2r = W2.reshape(1, H)
    b2s = b2.reshape(1, 1)
    acts, vals = _tc_call(node_embed, gemb3, mask3, W1, b1r, w2r, b2s)
    return (acts.reshape(B).astype(prefix_sum.dtype), vals.reshape(B))


# speedup vs baseline: 153.4671x; 153.4671x over previous
import jax
import jax.numpy as jnp
import numpy as np
from jax import lax
from jax.experimental import pallas as pl
from jax.experimental.pallas import tpu as pltpu
from jax.experimental.pallas import tpu_sc as plsc

B = 256
NPG = 4096
N = B * NPG
D = 64
H = 128
NB = 65536
FMIN = float(np.finfo(np.float32).min)
FMAX = float(np.finfo(np.float32).max)

NWORKERS = 32
REGION = N // NWORKERS


def _sc_mask_body(banned_ref, fill_ref, out_ref, idx_v, region_v):
    cid = lax.axis_index("c")
    sid = lax.axis_index("s")
    wid = sid * 2 + cid
    lo = wid * REGION
    pltpu.sync_copy(fill_ref, region_v)
    pltpu.sync_copy(banned_ref, idx_v)
    fminv = jnp.full((16,), FMIN, jnp.float32)

    def body(i, carry):
        v = idx_v[pl.ds(i * 16, 16)]
        lcl = v - lo
        m = (lcl >= 0) & (lcl < REGION)
        lc = jnp.clip(lcl, 0, REGION - 1)
        plsc.store_scatter(region_v, [lc], fminv, mask=m)
        return carry

    lax.fori_loop(0, NB // 16, body, 0, unroll=4)
    pltpu.sync_copy(region_v, out_ref.at[pl.ds(lo, REGION)])


def _sc_mask(banned_idx, fill):
    return pl.kernel(
        _sc_mask_body,
        out_type=jax.ShapeDtypeStruct((N,), jnp.float32),
        mesh=plsc.VectorSubcoreMesh(core_axis_name="c", subcore_axis_name="s"),
        scratch_types=[
            pltpu.VMEM((NB,), jnp.int32),
            pltpu.VMEM((REGION,), jnp.float32),
        ],
        compiler_params=pltpu.CompilerParams(needs_layout_passes=False),
    )(banned_idx, fill)


def _tc_body(x_ref, g_ref, m_ref, w1_ref, b1_ref, w2_ref, b2_ref,
             act_ref, val_ref):
    x = x_ref[...]
    g = g_ref[0]
    c = jnp.concatenate([x, jnp.broadcast_to(g, (NPG, D))], axis=1)
    h = lax.dot_general(c, w1_ref[...], (((1,), (0,)), ((), ())),
                        preferred_element_type=jnp.float32)
    h = jnp.maximum(h + b1_ref[...], 0.0)
    q = lax.dot_general(w2_ref[...], h, (((1,), (1,)), ((), ())),
                        preferred_element_type=jnp.float32)
    q = q + b2_ref[0, 0]
    qm = jnp.minimum(q, m_ref[0])
    vmax = jnp.max(qm)
    ids = lax.broadcasted_iota(jnp.int32, (1, NPG), 1)
    cand = jnp.where(qm == vmax, ids, NPG)
    amin = jnp.min(cand)
    act_ref[...] = jnp.reshape(amin, (1, 1, 1))
    val_ref[...] = jnp.reshape(vmax, (1, 1, 1))


def _tc_call(node_embed, gemb3, mask3, W1, b1r, w2r, b2s):
    acts, vals = pl.pallas_call(
        _tc_body,
        grid=(B,),
        in_specs=[
            pl.BlockSpec((NPG, D), lambda g: (g, 0)),
            pl.BlockSpec((1, 1, D), lambda g: (g, 0, 0)),
            pl.BlockSpec((1, 1, NPG), lambda g: (g, 0, 0)),
            pl.BlockSpec((2 * D, H), lambda g: (0, 0)),
            pl.BlockSpec((1, H), lambda g: (0, 0)),
            pl.BlockSpec((1, H), lambda g: (0, 0)),
            pl.BlockSpec(memory_space=pltpu.SMEM),
        ],
        out_specs=[
            pl.BlockSpec((1, 1, 1), lambda g: (g, 0, 0)),
            pl.BlockSpec((1, 1, 1), lambda g: (g, 0, 0)),
        ],
        out_shape=[
            jax.ShapeDtypeStruct((B, 1, 1), jnp.int32),
            jax.ShapeDtypeStruct((B, 1, 1), jnp.float32),
        ],
        compiler_params=pltpu.CompilerParams(
            dimension_semantics=("arbitrary",),
        ),
    )(node_embed, gemb3, mask3, W1, b1r, w2r, b2s)
    return acts, vals


def kernel(node_embed, graph_embed, prefix_sum, banned_idx, W1, b1, W2, b2):
    fill = jnp.full((REGION,), FMAX, jnp.float32)
    mask = _sc_mask(banned_idx, fill)
    mask3 = mask.reshape(B, 1, NPG)
    gemb3 = graph_embed.reshape(B, 1, D)
    b1r = b1.reshape(1, H)
    w2r = W2.reshape(1, H)
    b2s = b2.reshape(1, 1)
    acts, vals = _tc_call(node_embed, gemb3, mask3, W1, b1r, w2r, b2s)
    return (acts.reshape(B).astype(prefix_sum.dtype), vals.reshape(B))


# speedup vs baseline: 306.8067x; 1.9992x over previous
import jax
import jax.numpy as jnp
import numpy as np
from jax import lax
from jax.experimental import pallas as pl
from jax.experimental.pallas import tpu as pltpu
from jax.experimental.pallas import tpu_sc as plsc

B = 256
NPG = 4096
N = B * NPG
D = 64
H = 128
NB = 65536
FMIN = float(np.finfo(np.float32).min)
FMAX = float(np.finfo(np.float32).max)

NWORKERS = 32
REGION = N // NWORKERS


def _sc_mask_body(banned_ref, fill_ref, out_ref, idx_v, region_v):
    cid = lax.axis_index("c")
    sid = lax.axis_index("s")
    wid = sid * 2 + cid
    lo = wid * REGION
    pltpu.sync_copy(fill_ref, region_v)
    pltpu.sync_copy(banned_ref, idx_v)
    fminv = jnp.full((16,), FMIN, jnp.float32)

    def body(i, carry):
        v = idx_v[pl.ds(i * 16, 16)]
        lcl = v - lo
        m = (lcl >= 0) & (lcl < REGION)
        lc = jnp.clip(lcl, 0, REGION - 1)
        plsc.store_scatter(region_v, [lc], fminv, mask=m)
        return carry

    lax.fori_loop(0, NB // 16, body, 0, unroll=4)
    pltpu.sync_copy(region_v, out_ref.at[pl.ds(lo, REGION)])


def _sc_mask(banned_idx, fill):
    return pl.kernel(
        _sc_mask_body,
        out_type=jax.ShapeDtypeStruct((N,), jnp.float32),
        mesh=plsc.VectorSubcoreMesh(core_axis_name="c", subcore_axis_name="s"),
        scratch_types=[
            pltpu.VMEM((NB,), jnp.int32),
            pltpu.VMEM((REGION,), jnp.float32),
        ],
        compiler_params=pltpu.CompilerParams(needs_layout_passes=False),
    )(banned_idx, fill)


def _tc_body(xt_ref, gt_ref, m_ref, w1t_ref, b1_ref, w2_ref, b2_ref,
             act_ref, val_ref):
    xt = xt_ref[...]
    gcol = gt_ref[0]
    ct = jnp.concatenate(
        [xt, jnp.broadcast_to(gcol, (D, NPG))], axis=0)
    ht = lax.dot_general(w1t_ref[...], ct, (((1,), (0,)), ((), ())),
                         preferred_element_type=jnp.float32)
    ht = jnp.maximum(ht + b1_ref[...], 0.0)
    q = lax.dot_general(w2_ref[...], ht, (((1,), (0,)), ((), ())),
                        preferred_element_type=jnp.float32)
    q = q + b2_ref[0, 0]
    qm = jnp.minimum(q, m_ref[0])
    vmax = jnp.max(qm)
    ids = lax.broadcasted_iota(jnp.int32, (1, NPG), 1)
    cand = jnp.where(qm == vmax, ids, NPG)
    amin = jnp.min(cand)
    act_ref[...] = jnp.reshape(amin, (1, 1, 1))
    val_ref[...] = jnp.reshape(vmax, (1, 1, 1))


def _tc_call(node_embed_t, gemb_t, mask3, W1t, b1c, w2r, b2s):
    acts, vals = pl.pallas_call(
        _tc_body,
        grid=(B,),
        in_specs=[
            pl.BlockSpec((D, NPG), lambda g: (0, g)),
            pl.BlockSpec((1, D, 1), lambda g: (g, 0, 0)),
            pl.BlockSpec((1, 1, NPG), lambda g: (g, 0, 0)),
            pl.BlockSpec((H, 2 * D), lambda g: (0, 0)),
            pl.BlockSpec((H, 1), lambda g: (0, 0)),
            pl.BlockSpec((1, H), lambda g: (0, 0)),
            pl.BlockSpec(memory_space=pltpu.SMEM),
        ],
        out_specs=[
            pl.BlockSpec((1, 1, 1), lambda g: (g, 0, 0)),
            pl.BlockSpec((1, 1, 1), lambda g: (g, 0, 0)),
        ],
        out_shape=[
            jax.ShapeDtypeStruct((B, 1, 1), jnp.int32),
            jax.ShapeDtypeStruct((B, 1, 1), jnp.float32),
        ],
        compiler_params=pltpu.CompilerParams(
            dimension_semantics=("arbitrary",),
        ),
    )(node_embed_t, gemb_t, mask3, W1t, b1c, w2r, b2s)
    return acts, vals


def kernel(node_embed, graph_embed, prefix_sum, banned_idx, W1, b1, W2, b2):
    fill = jnp.full((REGION,), FMAX, jnp.float32)
    mask = _sc_mask(banned_idx, fill)
    mask3 = mask.reshape(B, 1, NPG)
    xt = node_embed.T
    gt = graph_embed.reshape(B, D, 1)
    W1t = W1.T
    b1c = b1.reshape(H, 1)
    w2r = W2.reshape(1, H)
    b2s = b2.reshape(1, 1)
    acts, vals = _tc_call(xt, gt, mask3, W1t, b1c, w2r, b2s)
    return (acts.reshape(B).astype(prefix_sum.dtype), vals.reshape(B))
